# baseline (speedup 1.0000x reference)
import jax
import jax.numpy as jnp
from jax.experimental import pallas as pl


def _body(idx_ref, tab_ref, w1_ref, b1_ref, w2_ref, b2_ref, out_ref):
    idx = idx_ref[:, :]
    n_rows = tab_ref.shape[0]
    cols = jax.lax.broadcasted_iota(jnp.int32, (idx.shape[0], n_rows), 1)
    onehot = (idx == cols).astype(jnp.float32)
    prompt = jnp.dot(onehot, tab_ref[:, :], preferred_element_type=jnp.float32)
    h = jnp.tanh(
        jnp.dot(prompt, w1_ref[:, :], preferred_element_type=jnp.float32)
        + b1_ref[:, :]
    )
    out = jnp.dot(h, w2_ref[:, :], preferred_element_type=jnp.float32) + b2_ref[:, :]
    out_ref[:, :, :] = jnp.broadcast_to(out[None], out_ref.shape)


def kernel(tokens, batch_size, pre_prompt, embd_table, W1, b1, W2, b2):
    B = tokens.shape[0]
    P = pre_prompt.shape[0]
    D = embd_table.shape[1]
    return pl.pallas_call(
        _body,
        out_shape=jax.ShapeDtypeStruct((B, P, D), jnp.float32),
    )(
        pre_prompt.reshape(P, 1),
        embd_table,
        W1,
        b1.reshape(1, -1),
        W2,
        b2.reshape(1, -1),
    )
